# baseline (speedup 1.0000x reference)
import functools

import jax
import jax.numpy as jnp
from jax import lax
from jax.experimental import pallas as pl
from jax.experimental.pallas import tpu as pltpu
from jax.experimental.pallas import tpu_sc as plsc

N_TASKS = 1024
N_SPLITS = 8
N_SKILLS = 64
D = N_SPLITS * N_SKILLS
B = 16384
EPS = 1e-12

_NC = 2
_NS = 16
_NW = _NC * _NS

_K = 1
_BC = B // _K
_B_PER_W = _BC // _NW
_CH = 64
_NCH = _B_PER_W // _CH
_NBUF = 2

_SUB = 512
_NSUB = _BC // _SUB


def _gather_body(table_hbm, ids_hbm, out_hbm, idx_v,
                 rb0, rb1, g0, g1, s0, s1):
    rbufs = [rb0, rb1]
    gsems = [g0, g1]
    ssems = [s0, s1]

    wid = lax.axis_index("s") * _NC + lax.axis_index("c")
    base = wid * _B_PER_W
    pltpu.sync_copy(ids_hbm.at[wid], idx_v)

    def fire_gather(ch):
        b = ch % _NBUF
        return pltpu.async_copy(table_hbm.at[idx_v.at[ch]], rbufs[b], gsems[b])

    gops = [None] * _NCH
    sops = [None] * _NCH
    for ch in range(_NBUF):
        gops[ch] = fire_gather(ch)
    for ch in range(_NCH):
        b = ch % _NBUF
        gops[ch].wait()
        if ch >= 1:
            sops[ch - 1].wait()
            nxt = ch - 1 + _NBUF
            if nxt < _NCH:
                gops[nxt] = fire_gather(nxt)
        sops[ch] = pltpu.async_copy(
            rbufs[b], out_hbm.at[pl.ds(base + ch * _CH, _CH)], ssems[b])
    sops[_NCH - 1].wait()


_mesh = plsc.VectorSubcoreMesh(core_axis_name="c", subcore_axis_name="s")

_gather = functools.partial(
    pl.kernel,
    mesh=_mesh,
    out_type=jax.ShapeDtypeStruct((_BC, D), jnp.float32),
    scratch_types=[
        pltpu.VMEM((_NCH, _CH), jnp.int32),
        pltpu.VMEM((_CH, D), jnp.float32),
        pltpu.VMEM((_CH, D), jnp.float32),
        pltpu.SemaphoreType.DMA,
        pltpu.SemaphoreType.DMA,
        pltpu.SemaphoreType.DMA,
        pltpu.SemaphoreType.DMA,
    ],
)(_gather_body)


def _norm_body(big_ref, x_ref, out_ref):
    del big_ref
    x = x_ref[...]
    s = 1.0 / (1.0 + jnp.exp(-x))
    for g in range(N_SPLITS):
        grp = s[:, g * N_SKILLS:(g + 1) * N_SKILLS]
        tot = jnp.sum(grp, axis=1, keepdims=True) + EPS
        out_ref[:, g, :] = grp * (1.0 / tot)


def _make_norm_chunk(chunk_idx):
    return pl.pallas_call(
        _norm_body,
        grid=(_NSUB,),
        in_specs=[
            pl.BlockSpec(memory_space=pl.ANY),
            pl.BlockSpec((_SUB, D), lambda i: (i, 0)),
        ],
        out_specs=pl.BlockSpec(
            (_SUB, N_SPLITS, N_SKILLS),
            lambda i, c=chunk_idx: (c * _NSUB + i, 0, 0)),
        out_shape=jax.ShapeDtypeStruct((B, N_SPLITS, N_SKILLS), jnp.float32),
        input_output_aliases={0: 0},
    )


_norm_chunks = [_make_norm_chunk(k) for k in range(_K)]

_norm_first = pl.pallas_call(
    _norm_body,
    grid=(_NSUB,),
    in_specs=[
        pl.BlockSpec(memory_space=pl.ANY),
        pl.BlockSpec((_SUB, D), lambda i: (i, 0)),
    ],
    out_specs=pl.BlockSpec(
        (_SUB, N_SPLITS, N_SKILLS), lambda i: (i, 0, 0)),
    out_shape=jax.ShapeDtypeStruct((B, N_SPLITS, N_SKILLS), jnp.float32),
)


@jax.jit
def kernel(module_logits, task_ids):
    ids = task_ids.astype(jnp.int32).reshape(_K, _NW, _NCH, _CH)
    chunks = [_gather(module_logits, ids[k]) for k in range(_K)]
    dummy = jnp.zeros((1, 1), jnp.float32)
    big = _norm_first(dummy, chunks[0])
    for k in range(1, _K):
        big = _norm_chunks[k](big, chunks[k])
    return big


# speedup vs baseline: 1.7396x; 1.7396x over previous
import functools

import jax
import jax.numpy as jnp
from jax import lax
from jax.experimental import pallas as pl
from jax.experimental.pallas import tpu as pltpu
from jax.experimental.pallas import tpu_sc as plsc

N_TASKS = 1024
N_SPLITS = 8
N_SKILLS = 64
D = N_SPLITS * N_SKILLS
B = 16384
EPS = 1e-12

_NC = 2
_NS = 16
_NW = _NC * _NS

_B_PER_W = B // _NW
_CH = 64
_NCH = _B_PER_W // _CH
_NBUF = 2


def _norm_body(table_ref, out_ref):
    x = table_ref[...]
    s = 1.0 / (1.0 + jnp.exp(-x))
    for g in range(N_SPLITS):
        sl = slice(g * N_SKILLS, (g + 1) * N_SKILLS)
        grp = s[:, sl]
        tot = jnp.sum(grp, axis=1, keepdims=True) + EPS
        out_ref[:, sl] = grp * (1.0 / tot)


_normalize = pl.pallas_call(
    _norm_body,
    out_shape=jax.ShapeDtypeStruct((N_TASKS, D), jnp.float32),
)


def _gather_body(norm_hbm, ids_hbm, out_hbm, idx_v,
                 rb0, rb1, g0, g1, s0, s1):
    rbufs = [rb0, rb1]
    gsems = [g0, g1]
    ssems = [s0, s1]

    wid = lax.axis_index("s") * _NC + lax.axis_index("c")
    base = wid * _B_PER_W
    pltpu.sync_copy(ids_hbm.at[wid], idx_v)

    def fire_gather(ch):
        b = ch % _NBUF
        return pltpu.async_copy(norm_hbm.at[idx_v.at[ch]], rbufs[b], gsems[b])

    gops = [None] * _NCH
    sops = [None] * _NCH
    for ch in range(_NBUF):
        gops[ch] = fire_gather(ch)
    for ch in range(_NCH):
        b = ch % _NBUF
        gops[ch].wait()
        if ch >= 1:
            sops[ch - 1].wait()
            nxt = ch - 1 + _NBUF
            if nxt < _NCH:
                gops[nxt] = fire_gather(nxt)
        sops[ch] = pltpu.async_copy(
            rbufs[b], out_hbm.at[pl.ds(base + ch * _CH, _CH)], ssems[b])
    sops[_NCH - 1].wait()


_mesh = plsc.VectorSubcoreMesh(core_axis_name="c", subcore_axis_name="s")

_gather = functools.partial(
    pl.kernel,
    mesh=_mesh,
    out_type=jax.ShapeDtypeStruct((B, D), jnp.float32),
    scratch_types=[
        pltpu.VMEM((_NCH, _CH), jnp.int32),
        pltpu.VMEM((_CH, D), jnp.float32),
        pltpu.VMEM((_CH, D), jnp.float32),
        pltpu.SemaphoreType.DMA,
        pltpu.SemaphoreType.DMA,
        pltpu.SemaphoreType.DMA,
        pltpu.SemaphoreType.DMA,
    ],
)(_gather_body)


@jax.jit
def kernel(module_logits, task_ids):
    norm = _normalize(module_logits)
    ids = task_ids.astype(jnp.int32).reshape(_NW, _NCH, _CH)
    out = _gather(norm, ids).reshape(B, N_SPLITS, N_SKILLS)
    return out * (1.0 / (jnp.sum(out, axis=-1, keepdims=True) + EPS))
